# baseline (speedup 1.0000x reference)
import functools

import jax
import jax.numpy as jnp
from jax import lax
from jax.experimental import pallas as pl
from jax.experimental.pallas import tpu as pltpu
from jax.experimental.pallas import tpu_sc as plsc

_TOKENS = 32768
_DIM = 768
_E = 8
_EPS = 1e-6
_CAP = float(_TOKENS)
_LANES = 128
_TBLK = 1024
_GRID = _TOKENS // _TBLK
_NC = 2
_NS = 16
_NW = _NC * _NS
_CHUNK = _TOKENS // _NW
_FLAT = _CHUNK * _E
_VL = 16


def _router_tc(x_ref, w_ref, b_ref, masked_ref, part_ref):
    y = jnp.dot(x_ref[...], w_ref[...],
                preferred_element_type=jnp.float32) + b_ref[...]
    yt = y[:, :_E].T
    m = jnp.max(yt, axis=0, keepdims=True)
    z = jnp.sum(jnp.exp(yt - m), axis=0, keepdims=True)
    s = 1.0 / z
    eidx = lax.broadcasted_iota(jnp.int32, (_E, 1), 0).astype(jnp.float32)
    key = jnp.where(yt >= m, eidx, float(_E))
    emin = jnp.min(key, axis=0, keepdims=True)
    masked = jnp.where(eidx == emin, s, 0.0)
    masked_ref[...] = masked
    part_ref[...] = jnp.sum(masked, axis=1)[None, :, None]


_tc_call = pl.pallas_call(
    _router_tc,
    grid=(_GRID,),
    in_specs=[
        pl.BlockSpec((_TBLK, _DIM), lambda i: (i, 0)),
        pl.BlockSpec((_DIM, _LANES), lambda i: (0, 0)),
        pl.BlockSpec((1, _LANES), lambda i: (0, 0)),
    ],
    out_specs=[
        pl.BlockSpec((_E, _TBLK), lambda i: (0, i)),
        pl.BlockSpec((1, _E, 1), lambda i: (i, 0, 0)),
    ],
    out_shape=[
        jax.ShapeDtypeStruct((_E, _TOKENS), jnp.float32),
        jax.ShapeDtypeStruct((_GRID, _E, 1), jnp.float32),
    ],
)


def _norm_sc_body(masked_hbm, part_hbm, out_hbm, v_v, p_v, o_v):
    wid = lax.axis_index("s") * _NC + lax.axis_index("c")
    tbase = wid * _CHUNK
    for e in range(_E):
        pltpu.sync_copy(masked_hbm.at[pl.ds(e * _TOKENS + tbase, _CHUNK)],
                        v_v.at[pl.ds(e * _CHUNK, _CHUNK)])
    pltpu.sync_copy(part_hbm, p_v)
    acc = jnp.zeros((_VL,), jnp.float32)
    for k in range(_GRID * _E // _VL):
        acc = acc + p_v[pl.ds(k * _VL, _VL)]
    perm = lax.iota(jnp.int32, _VL) ^ _E
    swapped = lax.gather(
        acc, perm[:, None],
        lax.GatherDimensionNumbers(offset_dims=(), collapsed_slice_dims=(0,),
                                   start_index_map=(0,)),
        slice_sizes=(1,), mode=lax.GatherScatterMode.PROMISE_IN_BOUNDS)
    denom = acc + swapped
    gvec = _CAP / (denom + _EPS)
    tok8 = lax.iota(jnp.int32, _VL) * _E
    for e in range(_E):
        ge = gvec[e]
        for k in range(_CHUNK // _VL):
            sv = v_v[pl.ds(e * _CHUNK + k * _VL, _VL)]
            idx = tok8 + (k * _VL * _E + e)
            plsc.store_scatter(o_v, [idx], sv * ge)
    pltpu.sync_copy(o_v, out_hbm.at[pl.ds(tbase * _E, _FLAT)])


@functools.lru_cache(maxsize=1)
def _get_sc_call():
    return pl.kernel(
        _norm_sc_body,
        out_type=jax.ShapeDtypeStruct((_TOKENS * _E,), jnp.float32),
        mesh=plsc.VectorSubcoreMesh(
            core_axis_name="c", subcore_axis_name="s",
            num_cores=_NC, num_subcores=_NS,
        ),
        compiler_params=pltpu.CompilerParams(needs_layout_passes=False),
        scratch_types=[
            pltpu.VMEM((_FLAT,), jnp.float32),
            pltpu.VMEM((_GRID * _E,), jnp.float32),
            pltpu.VMEM((_FLAT,), jnp.float32),
        ],
    )


def kernel(x, W, b):
    w_pad = jnp.zeros((_DIM, _LANES), jnp.float32).at[:, :_E].set(W)
    b_pad = jnp.zeros((1, _LANES), jnp.float32).at[0, :_E].set(b)
    masked, part = _tc_call(x, w_pad, b_pad)
    out = _get_sc_call()(masked.reshape(-1), part.reshape(-1))
    return out.reshape(_TOKENS, _E)


# speedup vs baseline: 1.0998x; 1.0998x over previous
import functools

import jax
import jax.numpy as jnp
from jax import lax
from jax.experimental import pallas as pl
from jax.experimental.pallas import tpu as pltpu
from jax.experimental.pallas import tpu_sc as plsc

_TOKENS = 32768
_DIM = 768
_E = 8
_EPS = 1e-6
_CAP = float(_TOKENS)
_LANES = 128
_TBLK = 8192
_GRID = _TOKENS // _TBLK
_NC = 2
_NS = 16
_NW = _NC * _NS
_CHUNK = _TOKENS // _NW
_FLAT = _CHUNK * _E
_VL = 16


def _router_tc(x_ref, w_ref, b_ref, masked_ref, part_ref):
    y = jnp.dot(x_ref[...], w_ref[...],
                preferred_element_type=jnp.float32) + b_ref[...]
    yt = y[:, :_E].T
    m = jnp.max(yt, axis=0, keepdims=True)
    z = jnp.sum(jnp.exp(yt - m), axis=0, keepdims=True)
    s = 1.0 / z
    eidx = lax.broadcasted_iota(jnp.int32, (_E, 1), 0).astype(jnp.float32)
    key = jnp.where(yt >= m, eidx, float(_E))
    emin = jnp.min(key, axis=0, keepdims=True)
    masked = jnp.where(eidx == emin, s, 0.0)
    masked_ref[...] = masked
    part_ref[...] = jnp.sum(masked, axis=1)[None, :, None]


_tc_call = pl.pallas_call(
    _router_tc,
    grid=(_GRID,),
    in_specs=[
        pl.BlockSpec((_TBLK, _DIM), lambda i: (i, 0)),
        pl.BlockSpec((_DIM, _LANES), lambda i: (0, 0)),
        pl.BlockSpec((1, _LANES), lambda i: (0, 0)),
    ],
    out_specs=[
        pl.BlockSpec((_E, _TBLK), lambda i: (0, i)),
        pl.BlockSpec((1, _E, 1), lambda i: (i, 0, 0)),
    ],
    out_shape=[
        jax.ShapeDtypeStruct((_E, _TOKENS), jnp.float32),
        jax.ShapeDtypeStruct((_GRID, _E, 1), jnp.float32),
    ],
)


def _norm_sc_body(masked_hbm, part_hbm, out_hbm, v_v, p_v, o_v):
    wid = lax.axis_index("s") * _NC + lax.axis_index("c")
    tbase = wid * _CHUNK
    for e in range(_E):
        pltpu.sync_copy(masked_hbm.at[pl.ds(e * _TOKENS + tbase, _CHUNK)],
                        v_v.at[pl.ds(e * _CHUNK, _CHUNK)])
    pltpu.sync_copy(part_hbm, p_v)
    acc = jnp.zeros((_VL,), jnp.float32)
    for k in range(_GRID * _E // _VL):
        acc = acc + p_v[pl.ds(k * _VL, _VL)]
    perm = lax.iota(jnp.int32, _VL) ^ _E
    swapped = lax.gather(
        acc, perm[:, None],
        lax.GatherDimensionNumbers(offset_dims=(), collapsed_slice_dims=(0,),
                                   start_index_map=(0,)),
        slice_sizes=(1,), mode=lax.GatherScatterMode.PROMISE_IN_BOUNDS)
    denom = acc + swapped
    gvec = _CAP / (denom + _EPS)
    tok8 = lax.iota(jnp.int32, _VL) * _E
    for e in range(_E):
        ge = gvec[e]
        for k in range(_CHUNK // _VL):
            sv = v_v[pl.ds(e * _CHUNK + k * _VL, _VL)]
            idx = tok8 + (k * _VL * _E + e)
            plsc.store_scatter(o_v, [idx], sv * ge)
    pltpu.sync_copy(o_v, out_hbm.at[pl.ds(tbase * _E, _FLAT)])


@functools.lru_cache(maxsize=1)
def _get_sc_call():
    return pl.kernel(
        _norm_sc_body,
        out_type=jax.ShapeDtypeStruct((_TOKENS * _E,), jnp.float32),
        mesh=plsc.VectorSubcoreMesh(
            core_axis_name="c", subcore_axis_name="s",
            num_cores=_NC, num_subcores=_NS,
        ),
        compiler_params=pltpu.CompilerParams(needs_layout_passes=False),
        scratch_types=[
            pltpu.VMEM((_FLAT,), jnp.float32),
            pltpu.VMEM((_GRID * _E,), jnp.float32),
            pltpu.VMEM((_FLAT,), jnp.float32),
        ],
    )


def kernel(x, W, b):
    w_pad = jnp.zeros((_DIM, _LANES), jnp.float32).at[:, :_E].set(W)
    b_pad = jnp.zeros((1, _LANES), jnp.float32).at[0, :_E].set(b)
    masked, part = _tc_call(x, w_pad, b_pad)
    out = _get_sc_call()(masked.reshape(-1), part.reshape(-1))
    return out.reshape(_TOKENS, _E)


# speedup vs baseline: 1.1953x; 1.0868x over previous
import functools

import jax
import jax.numpy as jnp
from jax import lax
from jax.experimental import pallas as pl
from jax.experimental.pallas import tpu as pltpu
from jax.experimental.pallas import tpu_sc as plsc

_TOKENS = 32768
_DIM = 768
_E = 8
_EPS = 1e-6
_CAP = float(_TOKENS)
_LANES = 128
_TBLK = 4096
_GRID = _TOKENS // _TBLK
_NC = 2
_NS = 16
_NW = _NC * _NS
_CHUNK = _TOKENS // _NW
_FLAT = _CHUNK * _E
_VL = 16


def _router_tc(x_ref, w_ref, b_ref, masked_ref, part_ref):
    y = jnp.dot(x_ref[...], w_ref[...],
                preferred_element_type=jnp.float32) + b_ref[...]
    yt = y[:, :_E].T
    m = jnp.max(yt, axis=0, keepdims=True)
    z = jnp.sum(jnp.exp(yt - m), axis=0, keepdims=True)
    s = 1.0 / z
    eidx = lax.broadcasted_iota(jnp.int32, (_E, 1), 0).astype(jnp.float32)
    key = jnp.where(yt >= m, eidx, float(_E))
    emin = jnp.min(key, axis=0, keepdims=True)
    masked = jnp.where(eidx == emin, s, 0.0)
    masked_ref[...] = masked
    part_ref[...] = jnp.sum(masked, axis=1)[None, :, None]


_tc_call = pl.pallas_call(
    _router_tc,
    grid=(_GRID,),
    in_specs=[
        pl.BlockSpec((_TBLK, _DIM), lambda i: (i, 0)),
        pl.BlockSpec((_DIM, _LANES), lambda i: (0, 0)),
        pl.BlockSpec((1, _LANES), lambda i: (0, 0)),
    ],
    out_specs=[
        pl.BlockSpec((_E, _TBLK), lambda i: (0, i)),
        pl.BlockSpec((1, _E, 1), lambda i: (i, 0, 0)),
    ],
    out_shape=[
        jax.ShapeDtypeStruct((_E, _TOKENS), jnp.float32),
        jax.ShapeDtypeStruct((_GRID, _E, 1), jnp.float32),
    ],
)


def _norm_sc_body(masked_hbm, part_hbm, out_hbm, v_v, p_v, o_v):
    wid = lax.axis_index("s") * _NC + lax.axis_index("c")
    tbase = wid * _CHUNK
    pltpu.sync_copy(masked_hbm.at[:, pl.ds(tbase, _CHUNK)], v_v)
    pltpu.sync_copy(part_hbm, p_v)
    acc = jnp.zeros((_VL,), jnp.float32)
    for k in range(_GRID * _E // _VL):
        acc = acc + p_v[pl.ds(k * _VL, _VL)]
    perm = lax.iota(jnp.int32, _VL) ^ _E
    swapped = lax.gather(
        acc, perm[:, None],
        lax.GatherDimensionNumbers(offset_dims=(), collapsed_slice_dims=(0,),
                                   start_index_map=(0,)),
        slice_sizes=(1,), mode=lax.GatherScatterMode.PROMISE_IN_BOUNDS)
    denom = acc + swapped
    gvec = _CAP / (denom + _EPS)
    tok8 = lax.iota(jnp.int32, _VL) * _E
    for e in range(_E):
        ge = gvec[e]
        for k in range(_CHUNK // _VL):
            sv = v_v[e, pl.ds(k * _VL, _VL)]
            idx = tok8 + (k * _VL * _E + e)
            plsc.store_scatter(o_v, [idx], sv * ge)
    pltpu.sync_copy(o_v, out_hbm.at[pl.ds(tbase * _E, _FLAT)])


@functools.lru_cache(maxsize=1)
def _get_sc_call():
    return pl.kernel(
        _norm_sc_body,
        out_type=jax.ShapeDtypeStruct((_TOKENS * _E,), jnp.float32),
        mesh=plsc.VectorSubcoreMesh(
            core_axis_name="c", subcore_axis_name="s",
            num_cores=_NC, num_subcores=_NS,
        ),
        compiler_params=pltpu.CompilerParams(needs_layout_passes=False),
        scratch_types=[
            pltpu.VMEM((_E, _CHUNK), jnp.float32),
            pltpu.VMEM((_GRID * _E,), jnp.float32),
            pltpu.VMEM((_FLAT,), jnp.float32),
        ],
    )


def kernel(x, W, b):
    w_pad = jnp.zeros((_DIM, _LANES), jnp.float32).at[:, :_E].set(W)
    b_pad = jnp.zeros((1, _LANES), jnp.float32).at[0, :_E].set(b)
    masked, part = _tc_call(x, w_pad, b_pad)
    out = _get_sc_call()(masked, part.reshape(-1))
    return out.reshape(_TOKENS, _E)


# speedup vs baseline: 1.2463x; 1.0426x over previous
import functools

import jax
import jax.numpy as jnp
from jax import lax
from jax.experimental import pallas as pl
from jax.experimental.pallas import tpu as pltpu
from jax.experimental.pallas import tpu_sc as plsc

_TOKENS = 32768
_DIM = 768
_E = 8
_EPS = 1e-6
_CAP = float(_TOKENS)
_LANES = 128
_TBLK = 4096
_GRID = _TOKENS // _TBLK
_NC = 2
_NS = 16
_NW = _NC * _NS
_CHUNK = _TOKENS // _NW
_FLAT = _CHUNK * _E
_VL = 16


def _router_tc(x_ref, w_ref, b_ref, masked_ref, part_ref):
    y = jnp.dot(x_ref[...], w_ref[...],
                preferred_element_type=jnp.float32) + b_ref[...]
    yt = y.T
    m = jnp.max(yt, axis=0, keepdims=True)
    z = jnp.sum(jnp.exp(yt - m), axis=0, keepdims=True)
    s = 1.0 / z
    eidx = lax.broadcasted_iota(jnp.int32, (_E, 1), 0).astype(jnp.float32)
    key = jnp.where(yt >= m, eidx, float(_E))
    emin = jnp.min(key, axis=0, keepdims=True)
    masked = jnp.where(eidx == emin, s, 0.0)
    masked_ref[...] = masked
    part_ref[...] = jnp.sum(masked, axis=1)[None, :, None]


_tc_call = pl.pallas_call(
    _router_tc,
    grid=(_GRID,),
    in_specs=[
        pl.BlockSpec((_TBLK, _DIM), lambda i: (i, 0)),
        pl.BlockSpec((_DIM, _E), lambda i: (0, 0)),
        pl.BlockSpec((1, _E), lambda i: (0, 0)),
    ],
    out_specs=[
        pl.BlockSpec((_E, _TBLK), lambda i: (0, i)),
        pl.BlockSpec((1, _E, 1), lambda i: (i, 0, 0)),
    ],
    out_shape=[
        jax.ShapeDtypeStruct((_E, _TOKENS), jnp.float32),
        jax.ShapeDtypeStruct((_GRID, _E, 1), jnp.float32),
    ],
)


def _norm_sc_body(masked_hbm, part_hbm, out_hbm, v_v, p_v, o_v):
    wid = lax.axis_index("s") * _NC + lax.axis_index("c")
    tbase = wid * _CHUNK
    pltpu.sync_copy(masked_hbm.at[:, pl.ds(tbase, _CHUNK)], v_v)
    pltpu.sync_copy(part_hbm, p_v)
    acc = jnp.zeros((_VL,), jnp.float32)
    for k in range(_GRID * _E // _VL):
        acc = acc + p_v[pl.ds(k * _VL, _VL)]
    perm = lax.iota(jnp.int32, _VL) ^ _E
    swapped = lax.gather(
        acc, perm[:, None],
        lax.GatherDimensionNumbers(offset_dims=(), collapsed_slice_dims=(0,),
                                   start_index_map=(0,)),
        slice_sizes=(1,), mode=lax.GatherScatterMode.PROMISE_IN_BOUNDS)
    denom = acc + swapped
    gvec = _CAP / (denom + _EPS)
    tok8 = lax.iota(jnp.int32, _VL) * _E
    for e in range(_E):
        ge = gvec[e]
        for k in range(_CHUNK // _VL):
            sv = v_v[e, pl.ds(k * _VL, _VL)]
            idx = tok8 + (k * _VL * _E + e)
            plsc.store_scatter(o_v, [idx], sv * ge)
    pltpu.sync_copy(o_v, out_hbm.at[pl.ds(tbase * _E, _FLAT)])


@functools.lru_cache(maxsize=1)
def _get_sc_call():
    return pl.kernel(
        _norm_sc_body,
        out_type=jax.ShapeDtypeStruct((_TOKENS * _E,), jnp.float32),
        mesh=plsc.VectorSubcoreMesh(
            core_axis_name="c", subcore_axis_name="s",
            num_cores=_NC, num_subcores=_NS,
        ),
        compiler_params=pltpu.CompilerParams(needs_layout_passes=False),
        scratch_types=[
            pltpu.VMEM((_E, _CHUNK), jnp.float32),
            pltpu.VMEM((_GRID * _E,), jnp.float32),
            pltpu.VMEM((_FLAT,), jnp.float32),
        ],
    )


def kernel(x, W, b):
    masked, part = _tc_call(x, W, b.reshape(1, _E))
    out = _get_sc_call()(masked, part.reshape(-1))
    return out.reshape(_TOKENS, _E)
